# Initial kernel scaffold; baseline (speedup 1.0000x reference)
import jax
import jax.numpy as jnp
from jax.experimental import pallas as pl


def kernel(x, adapt2_w, adapt2_b, adapt_w, adapt_b, qkv_w, qkv_b, out_w, out_b, ln1_g, ln1_b, lin1_w, lin1_b, lin2_w, lin2_b, ln2_g, ln2_b, fc_w, fc_b):
    raise NotImplementedError("write your pallas kernel here")


# speedup vs baseline: 2.2942x; 2.2942x over previous
import jax
import jax.numpy as jnp
from jax.experimental import pallas as pl
from jax.experimental.pallas import tpu as pltpu

S, E = 9, 631
D, FF, H, L, OUT = 512, 512, 8, 2, 512
HD = D // H
NB = 32
EPS = 1e-5
NEG = -1e30
SCALE = 1.0 / (HD ** 0.5)


def _ln(x, g, b):
    m = jnp.mean(x, axis=-1, keepdims=True)
    v = jnp.mean(jnp.square(x - m), axis=-1, keepdims=True)
    return (x - m) * jax.lax.rsqrt(v + EPS) * g + b


def _softmax(s):
    m = jnp.max(s, axis=-1, keepdims=True)
    e = jnp.exp(s - m)
    return e / jnp.sum(e, axis=-1, keepdims=True)


def _dot(a, b):
    return jnp.dot(a, b, preferred_element_type=jnp.float32)


def _dot_nt(a, b):
    return jax.lax.dot_general(a, b, (((1,), (1,)), ((), ())),
                               preferred_element_type=jnp.float32)


def _attention(qkv_parts, n_q, r, mask):
    q_all, k_all, v_all = qkv_parts
    ctxs = []
    for hh in range(H):
        sl = slice(hh * HD, (hh + 1) * HD)
        sc = _dot_nt(q_all[:, sl] * SCALE, k_all[:, sl])
        sc = jnp.where(mask, sc, NEG)
        ctxs.append(_dot(_softmax(sc), v_all[:, sl]))
    return jnp.concatenate(ctxs, axis=-1)


def _block_kernel(x_ref, aw_ref, ab_ref, w2r_ref, a2b_ref,
                  qkvw_ref, qkvb_ref, outw_ref, outb_ref,
                  ln1g_ref, ln1b_ref, l1w_ref, l1b_ref,
                  l2w_ref, l2b_ref, ln2g_ref, ln2b_ref,
                  fcw_ref, fcb_ref, o_ref):
    R = S * NB

    aw = aw_ref[...]
    ecols = jax.lax.broadcasted_iota(jnp.int32, (NB, E), 1)
    hs = []
    raw = None
    for s in range(S):
        oh = (x_ref[:, s:s + 1] == ecols).astype(jnp.float32)
        hs.append(_dot(oh, aw))
        contrib = _dot(oh, w2r_ref[s])
        raw = contrib if raw is None else raw + contrib
    h = jnp.concatenate(hs, axis=0) + ab_ref[...]
    raw = raw + a2b_ref[...]

    qkv = _dot(h, qkvw_ref[0]) + qkvb_ref[0]
    ri = jax.lax.broadcasted_iota(jnp.int32, (R, R), 0) & (NB - 1)
    ci = jax.lax.broadcasted_iota(jnp.int32, (R, R), 1) & (NB - 1)
    mask1 = ri == ci
    ctx = _attention((qkv[:, :D], qkv[:, D:2 * D], qkv[:, 2 * D:]),
                     R, R, mask1)
    h1 = _ln(h + _dot(ctx, outw_ref[0]) + outb_ref[0],
             ln1g_ref[0], ln1b_ref[0])
    ff = _dot(jax.nn.relu(_dot(h1, l1w_ref[0]) + l1b_ref[0]),
              l2w_ref[0]) + l2b_ref[0]
    h2 = _ln(h1 + ff, ln2g_ref[0], ln2b_ref[0])

    kv = _dot(h2, qkvw_ref[1][:, D:]) + qkvb_ref[1][:, D:]
    h20 = h2[:NB]
    q0 = _dot(h20, qkvw_ref[1][:, :D]) + qkvb_ref[1][:, :D]
    qi = jax.lax.broadcasted_iota(jnp.int32, (NB, R), 0)
    ki = jax.lax.broadcasted_iota(jnp.int32, (NB, R), 1) & (NB - 1)
    mask2 = qi == ki
    ctx0 = _attention((q0, kv[:, :D], kv[:, D:]), NB, R, mask2)
    g1 = _ln(h20 + _dot(ctx0, outw_ref[1]) + outb_ref[1],
             ln1g_ref[1], ln1b_ref[1])
    ff0 = _dot(jax.nn.relu(_dot(g1, l1w_ref[1]) + l1b_ref[1]),
               l2w_ref[1]) + l2b_ref[1]
    g2 = _ln(g1 + ff0, ln2g_ref[1], ln2b_ref[1])

    ov = _dot(g2, fcw_ref[...]) + fcb_ref[...]
    o_ref[...] = raw * (1.0 + jax.nn.relu(ov))


@jax.jit
def kernel(x, adapt2_w, adapt2_b, adapt_w, adapt_b, qkv_w, qkv_b, out_w,
           out_b, ln1_g, ln1_b, lin1_w, lin1_b, lin2_w, lin2_b, ln2_g,
           ln2_b, fc_w, fc_b):
    B, T, _ = x.shape
    N = B * T
    x2 = x.reshape(N, S).astype(jnp.int32)
    w2r = adapt2_w.reshape(S, E, OUT)

    full = lambda a: pl.BlockSpec(a.shape, lambda i: (0,) * a.ndim)
    operands = [
        x2, adapt_w, adapt_b.reshape(1, D), w2r, adapt2_b.reshape(1, OUT),
        qkv_w, qkv_b.reshape(L, 1, 3 * D), out_w, out_b.reshape(L, 1, D),
        ln1_g.reshape(L, 1, D), ln1_b.reshape(L, 1, D),
        lin1_w, lin1_b.reshape(L, 1, FF), lin2_w, lin2_b.reshape(L, 1, D),
        ln2_g.reshape(L, 1, D), ln2_b.reshape(L, 1, D),
        fc_w, fc_b.reshape(1, OUT),
    ]
    in_specs = [pl.BlockSpec((NB, S), lambda i: (i, 0))]
    in_specs += [full(a) for a in operands[1:]]

    out = pl.pallas_call(
        _block_kernel,
        grid=(N // NB,),
        in_specs=in_specs,
        out_specs=pl.BlockSpec((NB, OUT), lambda i: (i, 0)),
        out_shape=jax.ShapeDtypeStruct((N, OUT), jnp.float32),
        compiler_params=pltpu.CompilerParams(
            dimension_semantics=("parallel",),
            vmem_limit_bytes=100 * 1024 * 1024,
        ),
    )(*operands)
    return out.reshape(B, T, OUT)


# speedup vs baseline: 3.4171x; 1.4894x over previous
import jax
import jax.numpy as jnp
from jax.experimental import pallas as pl
from jax.experimental.pallas import tpu as pltpu

S, E = 9, 631
D, FF, H, L, OUT = 512, 512, 8, 2, 512
HD = D // H
NB = 32
EPS = 1e-5
NEG = -1e30
SCALE = 1.0 / (HD ** 0.5)


def _ln(x, g, b):
    m = jnp.mean(x, axis=-1, keepdims=True)
    v = jnp.mean(jnp.square(x - m), axis=-1, keepdims=True)
    return (x - m) * jax.lax.rsqrt(v + EPS) * g + b


def _softmax(s):
    m = jnp.max(s, axis=-1, keepdims=True)
    e = jnp.exp(s - m)
    return e / jnp.sum(e, axis=-1, keepdims=True)


def _dot(a, b):
    return jnp.dot(a, b, preferred_element_type=jnp.float32)


def _dot_nt(a, b):
    return jax.lax.dot_general(a, b, (((1,), (1,)), ((), ())),
                               preferred_element_type=jnp.float32)


def _attention(qkv_parts, n_q, r, mask):
    q_all, k_all, v_all = qkv_parts
    scs = []
    for hh in range(H):
        sl = slice(hh * HD, (hh + 1) * HD)
        sc = _dot_nt(q_all[:, sl], k_all[:, sl])
        scs.append(jnp.where(mask, sc, NEG))
    a = _softmax(jnp.concatenate(scs, axis=0))
    ctxs = []
    for hh in range(H):
        sl = slice(hh * HD, (hh + 1) * HD)
        ctxs.append(_dot(a[hh * n_q:(hh + 1) * n_q], v_all[:, sl]))
    return jnp.concatenate(ctxs, axis=-1)


def _block_kernel(x_ref, aw_ref, ab_ref, w2r_ref, a2b_ref,
                  qkvw_ref, qkvb_ref, outw_ref, outb_ref,
                  ln1g_ref, ln1b_ref, l1w_ref, l1b_ref,
                  l2w_ref, l2b_ref, ln2g_ref, ln2b_ref,
                  fcw_ref, fcb_ref, o_ref):
    R = S * NB

    aw = aw_ref[...]
    ecols = jax.lax.broadcasted_iota(jnp.int32, (NB, E), 1)
    hs = []
    raw = None
    for s in range(S):
        oh = (x_ref[:, s:s + 1] == ecols).astype(jnp.float32)
        hs.append(_dot(oh, aw))
        contrib = _dot(oh, w2r_ref[s])
        raw = contrib if raw is None else raw + contrib
    h = jnp.concatenate(hs, axis=0) + ab_ref[...]
    raw = raw + a2b_ref[...]

    qkv = _dot(h, qkvw_ref[0]) + qkvb_ref[0]
    ri = jax.lax.broadcasted_iota(jnp.int32, (R, R), 0) & (NB - 1)
    ci = jax.lax.broadcasted_iota(jnp.int32, (R, R), 1) & (NB - 1)
    mask1 = ri == ci
    ctx = _attention((qkv[:, :D], qkv[:, D:2 * D], qkv[:, 2 * D:]),
                     R, R, mask1)
    h1 = _ln(h + _dot(ctx, outw_ref[0]) + outb_ref[0],
             ln1g_ref[0], ln1b_ref[0])
    ff = _dot(jax.nn.relu(_dot(h1, l1w_ref[0]) + l1b_ref[0]),
              l2w_ref[0]) + l2b_ref[0]
    h2 = _ln(h1 + ff, ln2g_ref[0], ln2b_ref[0])

    kv = _dot(h2, qkvw_ref[1][:, D:]) + qkvb_ref[1][:, D:]
    h20 = h2[:NB]
    q0 = _dot(h20, qkvw_ref[1][:, :D]) + qkvb_ref[1][:, :D]
    qi = jax.lax.broadcasted_iota(jnp.int32, (NB, R), 0)
    ki = jax.lax.broadcasted_iota(jnp.int32, (NB, R), 1) & (NB - 1)
    mask2 = qi == ki
    ctx0 = _attention((q0, kv[:, :D], kv[:, D:]), NB, R, mask2)
    g1 = _ln(h20 + _dot(ctx0, outw_ref[1]) + outb_ref[1],
             ln1g_ref[1], ln1b_ref[1])
    ff0 = _dot(jax.nn.relu(_dot(g1, l1w_ref[1]) + l1b_ref[1]),
               l2w_ref[1]) + l2b_ref[1]
    g2 = _ln(g1 + ff0, ln2g_ref[1], ln2b_ref[1])

    ov = _dot(g2, fcw_ref[...]) + fcb_ref[...]
    o_ref[...] = raw * (1.0 + jax.nn.relu(ov))


@jax.jit
def kernel(x, adapt2_w, adapt2_b, adapt_w, adapt_b, qkv_w, qkv_b, out_w,
           out_b, ln1_g, ln1_b, lin1_w, lin1_b, lin2_w, lin2_b, ln2_g,
           ln2_b, fc_w, fc_b):
    B, T, _ = x.shape
    N = B * T
    x2 = x.reshape(N, S).astype(jnp.int32)
    w2r = adapt2_w.reshape(S, E, OUT)
    qscale = jnp.concatenate(
        [jnp.full((D,), SCALE, jnp.float32), jnp.ones((2 * D,), jnp.float32)])
    qkv_w = qkv_w * qscale
    qkv_b = qkv_b * qscale

    full = lambda a: pl.BlockSpec(a.shape, lambda i: (0,) * a.ndim)
    operands = [
        x2, adapt_w, adapt_b.reshape(1, D), w2r, adapt2_b.reshape(1, OUT),
        qkv_w, qkv_b.reshape(L, 1, 3 * D), out_w, out_b.reshape(L, 1, D),
        ln1_g.reshape(L, 1, D), ln1_b.reshape(L, 1, D),
        lin1_w, lin1_b.reshape(L, 1, FF), lin2_w, lin2_b.reshape(L, 1, D),
        ln2_g.reshape(L, 1, D), ln2_b.reshape(L, 1, D),
        fc_w, fc_b.reshape(1, OUT),
    ]
    in_specs = [pl.BlockSpec((NB, S), lambda i: (i, 0))]
    in_specs += [full(a) for a in operands[1:]]

    out = pl.pallas_call(
        _block_kernel,
        grid=(N // NB,),
        in_specs=in_specs,
        out_specs=pl.BlockSpec((NB, OUT), lambda i: (i, 0)),
        out_shape=jax.ShapeDtypeStruct((N, OUT), jnp.float32),
        compiler_params=pltpu.CompilerParams(
            dimension_semantics=("arbitrary",),
            vmem_limit_bytes=100 * 1024 * 1024,
        ),
    )(*operands)
    return out.reshape(B, T, OUT)


# speedup vs baseline: 3.7737x; 1.1044x over previous
import jax
import jax.numpy as jnp
from jax.experimental import pallas as pl
from jax.experimental.pallas import tpu as pltpu

S, E = 9, 631
D, FF, H, L, OUT = 512, 512, 8, 2, 512
HD = D // H
NB = 32
EPS = 1e-5
NEG = -1e30
SCALE = 1.0 / (HD ** 0.5)


def _ln(x, g, b):
    m = jnp.mean(x, axis=-1, keepdims=True)
    v = jnp.mean(jnp.square(x - m), axis=-1, keepdims=True)
    return (x - m) * jax.lax.rsqrt(v + EPS) * g + b


def _softmax(s):
    m = jnp.max(s, axis=-1, keepdims=True)
    e = jnp.exp(s - m)
    return e / jnp.sum(e, axis=-1, keepdims=True)


def _dot(a, b):
    return jnp.dot(a, b, preferred_element_type=jnp.float32)


def _dot_nt(a, b):
    return jax.lax.dot_general(a, b, (((1,), (1,)), ((), ())),
                               preferred_element_type=jnp.float32)


def _attention(qkv_parts, n_q, r, mask):
    q_all, k_all, v_all = qkv_parts
    scs = []
    for hh in range(H):
        sl = slice(hh * HD, (hh + 1) * HD)
        sc = _dot_nt(q_all[:, sl], k_all[:, sl])
        scs.append(jnp.where(mask, sc, NEG))
    a = _softmax(jnp.concatenate(scs, axis=0))
    ctxs = []
    for hh in range(H):
        sl = slice(hh * HD, (hh + 1) * HD)
        ctxs.append(_dot(a[hh * n_q:(hh + 1) * n_q], v_all[:, sl]))
    return jnp.concatenate(ctxs, axis=-1)


def _block_kernel(x_ref, aw_ref, ab_ref, w2r_ref, a2b_ref,
                  qkvw_ref, qkvb_ref, outw_ref, outb_ref,
                  ln1g_ref, ln1b_ref, l1w_ref, l1b_ref,
                  l2w_ref, l2b_ref, ln2g_ref, ln2b_ref,
                  fcw_ref, fcb_ref, o_ref):
    R = S * NB

    ecols = jax.lax.broadcasted_iota(jnp.int32, (NB, E), 1)
    ohs = [(x_ref[:, s:s + 1] == ecols).astype(jnp.float32) for s in range(S)]
    h = _dot(jnp.concatenate(ohs, axis=0), aw_ref[...]) + ab_ref[...]
    raw = _dot(jnp.concatenate(ohs, axis=1), w2r_ref[...]) + a2b_ref[...]

    qkv = _dot(h, qkvw_ref[0]) + qkvb_ref[0]
    ri = jax.lax.broadcasted_iota(jnp.int32, (R, R), 0) & (NB - 1)
    ci = jax.lax.broadcasted_iota(jnp.int32, (R, R), 1) & (NB - 1)
    mask1 = ri == ci
    ctx = _attention((qkv[:, :D], qkv[:, D:2 * D], qkv[:, 2 * D:]),
                     R, R, mask1)
    h1 = _ln(h + _dot(ctx, outw_ref[0]) + outb_ref[0],
             ln1g_ref[0], ln1b_ref[0])
    ff = _dot(jax.nn.relu(_dot(h1, l1w_ref[0]) + l1b_ref[0]),
              l2w_ref[0]) + l2b_ref[0]
    h2 = _ln(h1 + ff, ln2g_ref[0], ln2b_ref[0])

    kv = _dot(h2, qkvw_ref[1][:, D:]) + qkvb_ref[1][:, D:]
    h20 = h2[:NB]
    q0 = _dot(h20, qkvw_ref[1][:, :D]) + qkvb_ref[1][:, :D]
    qi = jax.lax.broadcasted_iota(jnp.int32, (NB, R), 0)
    ki = jax.lax.broadcasted_iota(jnp.int32, (NB, R), 1) & (NB - 1)
    mask2 = qi == ki
    ctx0 = _attention((q0, kv[:, :D], kv[:, D:]), NB, R, mask2)
    g1 = _ln(h20 + _dot(ctx0, outw_ref[1]) + outb_ref[1],
             ln1g_ref[1], ln1b_ref[1])
    ff0 = _dot(jax.nn.relu(_dot(g1, l1w_ref[1]) + l1b_ref[1]),
               l2w_ref[1]) + l2b_ref[1]
    g2 = _ln(g1 + ff0, ln2g_ref[1], ln2b_ref[1])

    ov = _dot(g2, fcw_ref[...]) + fcb_ref[...]
    o_ref[...] = raw * (1.0 + jax.nn.relu(ov))


@jax.jit
def kernel(x, adapt2_w, adapt2_b, adapt_w, adapt_b, qkv_w, qkv_b, out_w,
           out_b, ln1_g, ln1_b, lin1_w, lin1_b, lin2_w, lin2_b, ln2_g,
           ln2_b, fc_w, fc_b):
    B, T, _ = x.shape
    N = B * T
    x2 = x.reshape(N, S).astype(jnp.int32)
    w2r = adapt2_w
    qscale = jnp.concatenate(
        [jnp.full((D,), SCALE, jnp.float32), jnp.ones((2 * D,), jnp.float32)])
    qkv_w = qkv_w * qscale
    qkv_b = qkv_b * qscale

    full = lambda a: pl.BlockSpec(a.shape, lambda i: (0,) * a.ndim)
    operands = [
        x2, adapt_w, adapt_b.reshape(1, D), w2r, adapt2_b.reshape(1, OUT),
        qkv_w, qkv_b.reshape(L, 1, 3 * D), out_w, out_b.reshape(L, 1, D),
        ln1_g.reshape(L, 1, D), ln1_b.reshape(L, 1, D),
        lin1_w, lin1_b.reshape(L, 1, FF), lin2_w, lin2_b.reshape(L, 1, D),
        ln2_g.reshape(L, 1, D), ln2_b.reshape(L, 1, D),
        fc_w, fc_b.reshape(1, OUT),
    ]
    in_specs = [pl.BlockSpec((NB, S), lambda i: (i, 0))]
    in_specs += [full(a) for a in operands[1:]]

    out = pl.pallas_call(
        _block_kernel,
        grid=(N // NB,),
        in_specs=in_specs,
        out_specs=pl.BlockSpec((NB, OUT), lambda i: (i, 0)),
        out_shape=jax.ShapeDtypeStruct((N, OUT), jnp.float32),
        compiler_params=pltpu.CompilerParams(
            dimension_semantics=("arbitrary",),
            vmem_limit_bytes=100 * 1024 * 1024,
        ),
    )(*operands)
    return out.reshape(B, T, OUT)


# speedup vs baseline: 4.1733x; 1.1059x over previous
import jax
import jax.numpy as jnp
from jax.experimental import pallas as pl
from jax.experimental.pallas import tpu as pltpu

S, E = 9, 631
D, FF, H, L, OUT = 512, 512, 8, 2, 512
HD = D // H
NB = 64
EPS = 1e-5
NEG = -1e30
SCALE = 1.0 / (HD ** 0.5)


def _ln(x, g, b):
    m = jnp.mean(x, axis=-1, keepdims=True)
    v = jnp.mean(jnp.square(x - m), axis=-1, keepdims=True)
    return (x - m) * jax.lax.rsqrt(v + EPS) * g + b


def _softmax(s):
    m = jnp.max(s, axis=-1, keepdims=True)
    e = jnp.exp(s - m)
    return e / jnp.sum(e, axis=-1, keepdims=True)


def _dot(a, b):
    return jnp.dot(a, b, preferred_element_type=jnp.float32)


def _dot_nt(a, b):
    return jax.lax.dot_general(a, b, (((1,), (1,)), ((), ())),
                               preferred_element_type=jnp.float32)


def _attention(qkv_parts, n_q, r, mask):
    q_all, k_all, v_all = qkv_parts
    scs = []
    for hh in range(H):
        sl = slice(hh * HD, (hh + 1) * HD)
        sc = _dot_nt(q_all[:, sl], k_all[:, sl])
        scs.append(jnp.where(mask, sc, NEG))
    a = _softmax(jnp.concatenate(scs, axis=0))
    ctxs = []
    for hh in range(H):
        sl = slice(hh * HD, (hh + 1) * HD)
        ctxs.append(_dot(a[hh * n_q:(hh + 1) * n_q], v_all[:, sl]))
    return jnp.concatenate(ctxs, axis=-1)


def _block_kernel(x_ref, aw_ref, ab_ref, w2r_ref, a2b_ref,
                  qkvw_ref, qkvb_ref, outw_ref, outb_ref,
                  ln1g_ref, ln1b_ref, l1w_ref, l1b_ref,
                  l2w_ref, l2b_ref, ln2g_ref, ln2b_ref,
                  fcw_ref, fcb_ref, o_ref):
    R = S * NB

    ecols = jax.lax.broadcasted_iota(jnp.int32, (NB, E), 1)
    ohs = [(x_ref[:, s:s + 1] == ecols).astype(jnp.float32) for s in range(S)]
    h = _dot(jnp.concatenate(ohs, axis=0), aw_ref[...]) + ab_ref[...]
    raw = _dot(jnp.concatenate(ohs, axis=1), w2r_ref[...]) + a2b_ref[...]

    qkv = _dot(h, qkvw_ref[0]) + qkvb_ref[0]
    ri = jax.lax.broadcasted_iota(jnp.int32, (R, R), 0) & (NB - 1)
    ci = jax.lax.broadcasted_iota(jnp.int32, (R, R), 1) & (NB - 1)
    mask1 = ri == ci
    ctx = _attention((qkv[:, :D], qkv[:, D:2 * D], qkv[:, 2 * D:]),
                     R, R, mask1)
    h1 = _ln(h + _dot(ctx, outw_ref[0]) + outb_ref[0],
             ln1g_ref[0], ln1b_ref[0])
    ff = _dot(jax.nn.relu(_dot(h1, l1w_ref[0]) + l1b_ref[0]),
              l2w_ref[0]) + l2b_ref[0]
    h2 = _ln(h1 + ff, ln2g_ref[0], ln2b_ref[0])

    kv = _dot(h2, qkvw_ref[1][:, D:]) + qkvb_ref[1][:, D:]
    h20 = h2[:NB]
    q0 = _dot(h20, qkvw_ref[1][:, :D]) + qkvb_ref[1][:, :D]
    qi = jax.lax.broadcasted_iota(jnp.int32, (NB, R), 0)
    ki = jax.lax.broadcasted_iota(jnp.int32, (NB, R), 1) & (NB - 1)
    mask2 = qi == ki
    ctx0 = _attention((q0, kv[:, :D], kv[:, D:]), NB, R, mask2)
    g1 = _ln(h20 + _dot(ctx0, outw_ref[1]) + outb_ref[1],
             ln1g_ref[1], ln1b_ref[1])
    ff0 = _dot(jax.nn.relu(_dot(g1, l1w_ref[1]) + l1b_ref[1]),
               l2w_ref[1]) + l2b_ref[1]
    g2 = _ln(g1 + ff0, ln2g_ref[1], ln2b_ref[1])

    ov = _dot(g2, fcw_ref[...]) + fcb_ref[...]
    o_ref[...] = raw * (1.0 + jax.nn.relu(ov))


@jax.jit
def kernel(x, adapt2_w, adapt2_b, adapt_w, adapt_b, qkv_w, qkv_b, out_w,
           out_b, ln1_g, ln1_b, lin1_w, lin1_b, lin2_w, lin2_b, ln2_g,
           ln2_b, fc_w, fc_b):
    B, T, _ = x.shape
    N = B * T
    x2 = x.reshape(N, S).astype(jnp.int32)
    w2r = adapt2_w
    qscale = jnp.concatenate(
        [jnp.full((D,), SCALE, jnp.float32), jnp.ones((2 * D,), jnp.float32)])
    qkv_w = qkv_w * qscale
    qkv_b = qkv_b * qscale

    full = lambda a: pl.BlockSpec(a.shape, lambda i: (0,) * a.ndim)
    operands = [
        x2, adapt_w, adapt_b.reshape(1, D), w2r, adapt2_b.reshape(1, OUT),
        qkv_w, qkv_b.reshape(L, 1, 3 * D), out_w, out_b.reshape(L, 1, D),
        ln1_g.reshape(L, 1, D), ln1_b.reshape(L, 1, D),
        lin1_w, lin1_b.reshape(L, 1, FF), lin2_w, lin2_b.reshape(L, 1, D),
        ln2_g.reshape(L, 1, D), ln2_b.reshape(L, 1, D),
        fc_w, fc_b.reshape(1, OUT),
    ]
    in_specs = [pl.BlockSpec((NB, S), lambda i: (i, 0))]
    in_specs += [full(a) for a in operands[1:]]

    out = pl.pallas_call(
        _block_kernel,
        grid=(N // NB,),
        in_specs=in_specs,
        out_specs=pl.BlockSpec((NB, OUT), lambda i: (i, 0)),
        out_shape=jax.ShapeDtypeStruct((N, OUT), jnp.float32),
        compiler_params=pltpu.CompilerParams(
            dimension_semantics=("arbitrary",),
            vmem_limit_bytes=100 * 1024 * 1024,
        ),
    )(*operands)
    return out.reshape(B, T, OUT)


# speedup vs baseline: 5.3361x; 1.2786x over previous
import jax
import jax.numpy as jnp
from jax.experimental import pallas as pl
from jax.experimental.pallas import tpu as pltpu

S, E = 9, 631
D, FF, H, L, OUT = 512, 512, 8, 2, 512
HD = D // H
NB = 64
G = 16
NG = NB // G
GR = S * G
EPS = 1e-5
NEG = -1e30
SCALE = 1.0 / (HD ** 0.5)


def _ln(x, g, b):
    m = jnp.mean(x, axis=-1, keepdims=True)
    v = jnp.mean(jnp.square(x - m), axis=-1, keepdims=True)
    return (x - m) * jax.lax.rsqrt(v + EPS) * g + b


def _softmax(s):
    m = jnp.max(s, axis=-1, keepdims=True)
    e = jnp.exp(s - m)
    return e / jnp.sum(e, axis=-1, keepdims=True)


def _dot(a, b):
    return jnp.dot(a, b, preferred_element_type=jnp.float32)


def _dot_nt(a, b):
    return jax.lax.dot_general(a, b, (((1,), (1,)), ((), ())),
                               preferred_element_type=jnp.float32)


def _attention(q, kv_rows, n_q, maskb):
    scs = []
    for c in range(NG):
        qc = q[c * n_q:(c + 1) * n_q]
        kc = kv_rows[c * GR:(c + 1) * GR, :D]
        for hh in range(H):
            sl = slice(hh * HD, (hh + 1) * HD)
            scs.append(_dot_nt(qc[:, sl], kc[:, sl]) + maskb)
    a = _softmax(jnp.concatenate(scs, axis=0))
    groups = []
    for c in range(NG):
        vc = kv_rows[c * GR:(c + 1) * GR, D:]
        ctxs = []
        for hh in range(H):
            i = c * H + hh
            ctxs.append(_dot(a[i * n_q:(i + 1) * n_q],
                             vc[:, hh * HD:(hh + 1) * HD]))
        groups.append(jnp.concatenate(ctxs, axis=-1))
    return jnp.concatenate(groups, axis=0)


def _block_kernel(xg_ref, x_ref, m1_ref, m2_ref, aw_ref, ab_ref, w2r_ref,
                  a2b_ref, qkvw_ref, qkvb_ref, outw_ref, outb_ref,
                  ln1g_ref, ln1b_ref, l1w_ref, l1b_ref,
                  l2w_ref, l2b_ref, ln2g_ref, ln2b_ref,
                  fcw_ref, fcb_ref, o_ref):
    R = S * NB

    oh_all = (xg_ref[0] == jax.lax.broadcasted_iota(jnp.int32, (R, E), 1)
              ).astype(jnp.float32)
    h = _dot(oh_all, aw_ref[...]) + ab_ref[...]
    ecols = jax.lax.broadcasted_iota(jnp.int32, (NB, E), 1)
    ohs = [(x_ref[:, s:s + 1] == ecols).astype(jnp.float32) for s in range(S)]
    raw = _dot(jnp.concatenate(ohs, axis=1), w2r_ref[...]) + a2b_ref[...]

    m1 = m1_ref[...]
    m2 = m2_ref[...]

    qkv = _dot(h, qkvw_ref[0]) + qkvb_ref[0]
    ctx = _attention(qkv[:, :D], qkv[:, D:], GR, m1)
    h1 = _ln(h + _dot(ctx, outw_ref[0]) + outb_ref[0],
             ln1g_ref[0], ln1b_ref[0])
    ff = _dot(jax.nn.relu(_dot(h1, l1w_ref[0]) + l1b_ref[0]),
              l2w_ref[0]) + l2b_ref[0]
    h2 = _ln(h1 + ff, ln2g_ref[0], ln2b_ref[0])

    kv = _dot(h2, qkvw_ref[1][:, D:]) + qkvb_ref[1][:, D:]
    h20 = jnp.concatenate([h2[c * GR:c * GR + G] for c in range(NG)], axis=0)
    q0 = _dot(h20, qkvw_ref[1][:, :D]) + qkvb_ref[1][:, :D]
    ctx0 = _attention(q0, kv, G, m2)
    g1 = _ln(h20 + _dot(ctx0, outw_ref[1]) + outb_ref[1],
             ln1g_ref[1], ln1b_ref[1])
    ff0 = _dot(jax.nn.relu(_dot(g1, l1w_ref[1]) + l1b_ref[1]),
               l2w_ref[1]) + l2b_ref[1]
    g2 = _ln(g1 + ff0, ln2g_ref[1], ln2b_ref[1])

    ov = _dot(g2, fcw_ref[...]) + fcb_ref[...]
    o_ref[...] = raw * (1.0 + jax.nn.relu(ov))


@jax.jit
def kernel(x, adapt2_w, adapt2_b, adapt_w, adapt_b, qkv_w, qkv_b, out_w,
           out_b, ln1_g, ln1_b, lin1_w, lin1_b, lin2_w, lin2_b, ln2_g,
           ln2_b, fc_w, fc_b):
    B, T, _ = x.shape
    N = B * T
    x2 = x.reshape(N, S).astype(jnp.int32)
    xg = (x2.reshape(N // NB, NG, G, S).transpose(0, 1, 3, 2)
          .reshape(N // NB, S * NB, 1))
    qscale = jnp.concatenate(
        [jnp.full((D,), SCALE, jnp.float32), jnp.ones((2 * D,), jnp.float32)])
    qkv_w = qkv_w * qscale
    qkv_b = qkv_b * qscale
    ri = jax.lax.broadcasted_iota(jnp.int32, (GR, GR), 0) % G
    ci = jax.lax.broadcasted_iota(jnp.int32, (GR, GR), 1) % G
    m1 = jnp.where(ri == ci, 0.0, NEG).astype(jnp.float32)
    qi = jax.lax.broadcasted_iota(jnp.int32, (G, GR), 0)
    ki = jax.lax.broadcasted_iota(jnp.int32, (G, GR), 1) % G
    m2 = jnp.where(qi == ki, 0.0, NEG).astype(jnp.float32)

    full = lambda a: pl.BlockSpec(a.shape, lambda i: (0,) * a.ndim)
    operands = [
        xg, x2, m1, m2,
        adapt_w, adapt_b.reshape(1, D), adapt2_w, adapt2_b.reshape(1, OUT),
        qkv_w, qkv_b.reshape(L, 1, 3 * D), out_w, out_b.reshape(L, 1, D),
        ln1_g.reshape(L, 1, D), ln1_b.reshape(L, 1, D),
        lin1_w, lin1_b.reshape(L, 1, FF), lin2_w, lin2_b.reshape(L, 1, D),
        ln2_g.reshape(L, 1, D), ln2_b.reshape(L, 1, D),
        fc_w, fc_b.reshape(1, OUT),
    ]
    in_specs = [pl.BlockSpec((1, S * NB, 1), lambda i: (i, 0, 0)),
                pl.BlockSpec((NB, S), lambda i: (i, 0))]
    in_specs += [full(a) for a in operands[2:]]

    out = pl.pallas_call(
        _block_kernel,
        grid=(N // NB,),
        in_specs=in_specs,
        out_specs=pl.BlockSpec((NB, OUT), lambda i: (i, 0)),
        out_shape=jax.ShapeDtypeStruct((N, OUT), jnp.float32),
        compiler_params=pltpu.CompilerParams(
            dimension_semantics=("arbitrary",),
            vmem_limit_bytes=100 * 1024 * 1024,
        ),
    )(*operands)
    return out.reshape(B, T, OUT)


# speedup vs baseline: 6.0197x; 1.1281x over previous
import jax
import jax.numpy as jnp
from jax.experimental import pallas as pl
from jax.experimental.pallas import tpu as pltpu

S, E = 9, 631
D, FF, H, L, OUT = 512, 512, 8, 2, 512
HD = D // H
NB = 128
G = 16
NG = NB // G
GR = S * G
EPS = 1e-5
NEG = -1e30
SCALE = 1.0 / (HD ** 0.5)


def _ln(x, g, b):
    m = jnp.mean(x, axis=-1, keepdims=True)
    v = jnp.mean(x * x, axis=-1, keepdims=True) - m * m
    rs = jax.lax.rsqrt(v + EPS)
    return x * (rs * g) + (b - m * rs * g)


def _softmax(s):
    m = jnp.max(s, axis=-1, keepdims=True)
    e = jnp.exp(s - m)
    return e / jnp.sum(e, axis=-1, keepdims=True)


def _dot(a, b):
    return jnp.dot(a, b, preferred_element_type=jnp.float32)


def _dot_nt(a, b):
    return jax.lax.dot_general(a, b, (((1,), (1,)), ((), ())),
                               preferred_element_type=jnp.float32)


def _attention(q, kv_rows, n_q, maskb):
    scs = []
    for c in range(NG):
        qc = q[c * n_q:(c + 1) * n_q]
        kc = kv_rows[c * GR:(c + 1) * GR, :D]
        for hh in range(H):
            sl = slice(hh * HD, (hh + 1) * HD)
            scs.append(_dot_nt(qc[:, sl], kc[:, sl]) + maskb)
    a = _softmax(jnp.concatenate(scs, axis=0))
    groups = []
    for c in range(NG):
        vc = kv_rows[c * GR:(c + 1) * GR, D:]
        ctxs = []
        for hh in range(H):
            i = c * H + hh
            ctxs.append(_dot(a[i * n_q:(i + 1) * n_q],
                             vc[:, hh * HD:(hh + 1) * HD]))
        groups.append(jnp.concatenate(ctxs, axis=-1))
    return jnp.concatenate(groups, axis=0)


def _block_kernel(xg_ref, x_ref, m1_ref, m2_ref, aw_ref, ab_ref, w2r_ref,
                  a2b_ref, qkvw_ref, qkvb_ref, outw_ref, outb_ref,
                  ln1g_ref, ln1b_ref, l1w_ref, l1b_ref,
                  l2w_ref, l2b_ref, ln2g_ref, ln2b_ref,
                  fcw_ref, fcb_ref, o_ref):
    R = S * NB

    oh_all = (xg_ref[0] == jax.lax.broadcasted_iota(jnp.int32, (R, E), 1)
              ).astype(jnp.float32)
    h = _dot(oh_all, aw_ref[...]) + ab_ref[...]
    ecols = jax.lax.broadcasted_iota(jnp.int32, (NB, E), 1)
    ohs = [(x_ref[:, s:s + 1] == ecols).astype(jnp.float32) for s in range(S)]
    raw = _dot(jnp.concatenate(ohs, axis=1), w2r_ref[...]) + a2b_ref[...]

    m1 = m1_ref[...]
    m2 = m2_ref[...]

    qkv = _dot(h, qkvw_ref[0]) + qkvb_ref[0]
    ctx = _attention(qkv[:, :D], qkv[:, D:], GR, m1)
    h1 = _ln(h + _dot(ctx, outw_ref[0]) + outb_ref[0],
             ln1g_ref[0], ln1b_ref[0])
    ff = _dot(jax.nn.relu(_dot(h1, l1w_ref[0]) + l1b_ref[0]),
              l2w_ref[0]) + l2b_ref[0]
    h2 = _ln(h1 + ff, ln2g_ref[0], ln2b_ref[0])

    kv = _dot(h2, qkvw_ref[1][:, D:]) + qkvb_ref[1][:, D:]
    h20 = jnp.concatenate([h2[c * GR:c * GR + G] for c in range(NG)], axis=0)
    q0 = _dot(h20, qkvw_ref[1][:, :D]) + qkvb_ref[1][:, :D]
    ctx0 = _attention(q0, kv, G, m2)
    g1 = _ln(h20 + _dot(ctx0, outw_ref[1]) + outb_ref[1],
             ln1g_ref[1], ln1b_ref[1])
    ff0 = _dot(jax.nn.relu(_dot(g1, l1w_ref[1]) + l1b_ref[1]),
               l2w_ref[1]) + l2b_ref[1]
    g2 = _ln(g1 + ff0, ln2g_ref[1], ln2b_ref[1])

    ov = _dot(g2, fcw_ref[...]) + fcb_ref[...]
    o_ref[...] = raw * (1.0 + jax.nn.relu(ov))


@jax.jit
def kernel(x, adapt2_w, adapt2_b, adapt_w, adapt_b, qkv_w, qkv_b, out_w,
           out_b, ln1_g, ln1_b, lin1_w, lin1_b, lin2_w, lin2_b, ln2_g,
           ln2_b, fc_w, fc_b):
    B, T, _ = x.shape
    N = B * T
    x2 = x.reshape(N, S).astype(jnp.int32)
    xg = (x2.reshape(N // NB, NG, G, S).transpose(0, 1, 3, 2)
          .reshape(N // NB, S * NB, 1))
    qscale = jnp.concatenate(
        [jnp.full((D,), SCALE, jnp.float32), jnp.ones((2 * D,), jnp.float32)])
    qkv_w = qkv_w * qscale
    qkv_b = qkv_b * qscale
    ri = jax.lax.broadcasted_iota(jnp.int32, (GR, GR), 0) % G
    ci = jax.lax.broadcasted_iota(jnp.int32, (GR, GR), 1) % G
    m1 = jnp.where(ri == ci, 0.0, NEG).astype(jnp.float32)
    qi = jax.lax.broadcasted_iota(jnp.int32, (G, GR), 0)
    ki = jax.lax.broadcasted_iota(jnp.int32, (G, GR), 1) % G
    m2 = jnp.where(qi == ki, 0.0, NEG).astype(jnp.float32)

    full = lambda a: pl.BlockSpec(a.shape, lambda i: (0,) * a.ndim)
    operands = [
        xg, x2, m1, m2,
        adapt_w, adapt_b.reshape(1, D), adapt2_w, adapt2_b.reshape(1, OUT),
        qkv_w, qkv_b.reshape(L, 1, 3 * D), out_w, out_b.reshape(L, 1, D),
        ln1_g.reshape(L, 1, D), ln1_b.reshape(L, 1, D),
        lin1_w, lin1_b.reshape(L, 1, FF), lin2_w, lin2_b.reshape(L, 1, D),
        ln2_g.reshape(L, 1, D), ln2_b.reshape(L, 1, D),
        fc_w, fc_b.reshape(1, OUT),
    ]
    in_specs = [pl.BlockSpec((1, S * NB, 1), lambda i: (i, 0, 0)),
                pl.BlockSpec((NB, S), lambda i: (i, 0))]
    in_specs += [full(a) for a in operands[2:]]

    out = pl.pallas_call(
        _block_kernel,
        grid=(N // NB,),
        in_specs=in_specs,
        out_specs=pl.BlockSpec((NB, OUT), lambda i: (i, 0)),
        out_shape=jax.ShapeDtypeStruct((N, OUT), jnp.float32),
        compiler_params=pltpu.CompilerParams(
            dimension_semantics=("arbitrary",),
            vmem_limit_bytes=100 * 1024 * 1024,
        ),
    )(*operands)
    return out.reshape(B, T, OUT)
